# baseline (speedup 1.0000x reference)
import jax
import jax.numpy as jnp
from jax import lax
from jax.experimental import pallas as pl
from jax.experimental.pallas import tpu as pltpu
from jax.experimental.pallas import tpu_sc as plsc

NC = 2
NS = 16
NW = NC * NS
CHUNK = 80
DEGW = 128


def _sc_mesh():
    return plsc.VectorSubcoreMesh(
        core_axis_name="c", subcore_axis_name="s",
        num_cores=NC, num_subcores=NS)


def _fill_rows(buf, nrows, width, value):
    v16 = jnp.full((16,), value, jnp.float32)

    @pl.loop(0, nrows)
    def _(r):
        @pl.loop(0, width // 16)
        def _(j):
            buf[r, pl.ds(j * 16, 16)] = v16


ZC = 16


def _tile_rows(n_nodes, sid):
    lo = (n_nodes // NS) // ZC * ZC
    base = sid * lo
    nrows = lo + (n_nodes - NS * lo) * (sid == NS - 1)
    return base, nrows


GROUP = 8


def _sc_degree(dst2d, n_nodes, nchunk):
    nchunk_pad = dst2d.shape[0]
    kpw = nchunk_pad // NW
    ngrp = kpw // GROUP

    def body(dst_hbm, out_hbm, gdst0, gsem0, gdst1, gsem1,
             ones_v, zero_v, acc_sh, ssem):
        cid = lax.axis_index("c")
        sid = lax.axis_index("s")
        wid = cid * NS + sid
        _fill_rows(ones_v, CHUNK, DEGW, 1.0)
        _fill_rows(zero_v, ZC, DEGW, 0.0)
        base, nrows = _tile_rows(n_nodes, sid)

        @pl.loop(0, nrows // ZC)
        def _(k):
            pltpu.sync_copy(zero_v, acc_sh.at[pl.ds(base + k * ZC, ZC)])

        plsc.subcore_barrier()

        c00 = wid * kpw
        gbufs = ((gdst0, gsem0), (gdst1, gsem1))

        def fetch(g, b):
            gd, gsem = gbufs[b]

            @pl.when((g < ngrp) & (c00 + g * GROUP < nchunk))
            def _():
                pltpu.async_copy(dst_hbm.at[pl.ds(c00 + g * GROUP, GROUP)],
                                 gd, gsem)

        def wait_fetch(g, b):
            gd, gsem = gbufs[b]

            @pl.when((g < ngrp) & (c00 + g * GROUP < nchunk))
            def _():
                pltpu.make_async_copy(dst_hbm.at[pl.ds(0, GROUP)],
                                      gd, gsem).wait()

        fetch(0, 0)

        @pl.loop(0, ngrp)
        def _(g):
            b = lax.rem(g, 2)
            for bb in range(2):
                @pl.when(b == bb)
                def _():
                    gd, _gsem = gbufs[bb]
                    wait_fetch(g, bb)
                    fetch(g + 1, (bb + 1) % 2)
                    for j in range(GROUP):
                        c = c00 + g * GROUP + j

                        @pl.when(c < nchunk)
                        def _():
                            pltpu.async_copy(ones_v, acc_sh.at[gd.at[j]],
                                             ssem, add=True)
                    for j in range(GROUP):
                        c = c00 + g * GROUP + j

                        @pl.when(c < nchunk)
                        def _():
                            pltpu.make_async_copy(
                                ones_v, acc_sh.at[gd.at[j]], ssem).wait()

        plsc.subcore_barrier()

        @pl.loop(0, nrows // ZC)
        def _(k):
            pltpu.sync_copy(acc_sh.at[pl.ds(base + k * ZC, ZC)],
                            out_hbm.at[cid, pl.ds(base + k * ZC, ZC)])

    return pl.kernel(
        body,
        out_type=jax.ShapeDtypeStruct((NC, n_nodes, DEGW), jnp.float32),
        mesh=_sc_mesh(),
        scratch_types=[
            pltpu.VMEM((GROUP, CHUNK), jnp.int32),
            pltpu.SemaphoreType.DMA,
            pltpu.VMEM((GROUP, CHUNK), jnp.int32),
            pltpu.SemaphoreType.DMA,
            pltpu.VMEM((CHUNK, DEGW), jnp.float32),
            pltpu.VMEM((ZC, DEGW), jnp.float32),
            pltpu.VMEM_SHARED((n_nodes, DEGW), jnp.float32),
            pltpu.SemaphoreType.DMA,
        ],
    )(dst2d)


def _sc_propagate(table, src2d, dst2d, nchunk):
    n_nodes, width = table.shape
    nchunk_pad = src2d.shape[0]
    kpw = nchunk_pad // NW
    ngrp = kpw // GROUP

    def body(tab_hbm, src_hbm, dst_hbm, out_hbm,
             gsrc0, gdst0, gsem0, gsrc1, gdst1, gsem1,
             rows0, rsem0, ssem0, rows1, rsem1, ssem1,
             rows2, rsem2, ssem2, rows3, rsem3, ssem3,
             zero_v, acc_sh):
        cid = lax.axis_index("c")
        sid = lax.axis_index("s")
        wid = cid * NS + sid
        _fill_rows(zero_v, ZC, width, 0.0)
        base, nrows = _tile_rows(n_nodes, sid)

        @pl.loop(0, nrows // ZC)
        def _(k):
            pltpu.sync_copy(zero_v, acc_sh.at[pl.ds(base + k * ZC, ZC)])

        plsc.subcore_barrier()

        c00 = wid * kpw
        gbufs = ((gsrc0, gdst0, gsem0), (gsrc1, gdst1, gsem1))
        rbufs = ((rows0, rsem0, ssem0), (rows1, rsem1, ssem1),
                 (rows2, rsem2, ssem2), (rows3, rsem3, ssem3))
        RB = len(rbufs)

        def fetch(g, b):
            gs, gd, gsem = gbufs[b]

            @pl.when((g < ngrp) & (c00 + g * GROUP < nchunk))
            def _():
                pltpu.async_copy(src_hbm.at[pl.ds(c00 + g * GROUP, GROUP)],
                                 gs, gsem)
                pltpu.async_copy(dst_hbm.at[pl.ds(c00 + g * GROUP, GROUP)],
                                 gd, gsem)

        def wait_fetch(g, b):
            gs, gd, gsem = gbufs[b]

            @pl.when((g < ngrp) & (c00 + g * GROUP < nchunk))
            def _():
                pltpu.make_async_copy(src_hbm.at[pl.ds(0, GROUP)],
                                      gs, gsem).wait()
                pltpu.make_async_copy(dst_hbm.at[pl.ds(0, GROUP)],
                                      gd, gsem).wait()

        def drain_scatter(c, rb, gd_any):
            rows, _rsem, ssem = rbufs[rb]

            @pl.when((c >= c00) & (c < nchunk))
            def _():
                pltpu.make_async_copy(rows, acc_sh.at[gd_any.at[0]],
                                      ssem).wait()

        fetch(0, 0)

        @pl.loop(0, ngrp)
        def _(g):
            gb = lax.rem(g, 2)
            for bb in range(2):
                @pl.when(gb == bb)
                def _():
                    gs, gd, _gsem = gbufs[bb]
                    cg0 = c00 + g * GROUP
                    wait_fetch(g, bb)

                    def start(j):
                        c = cg0 + j
                        drain_scatter(c - RB, j % RB, gd)
                        rows, rsem, _ssem = rbufs[j % RB]

                        @pl.when(c < nchunk)
                        def _():
                            pltpu.async_copy(tab_hbm.at[gs.at[j]],
                                             rows, rsem)

                    def finish(j):
                        c = cg0 + j
                        rows, rsem, ssem = rbufs[j % RB]

                        @pl.when(c < nchunk)
                        def _():
                            pltpu.make_async_copy(tab_hbm.at[gs.at[j]],
                                                  rows, rsem).wait()
                            pltpu.async_copy(rows, acc_sh.at[gd.at[j]],
                                             ssem, add=True)

                    start(0)
                    start(1)
                    for j in range(GROUP):
                        if j + 2 < GROUP:
                            start(j + 2)
                        finish(j)
                        if j == 1:
                            fetch(g + 1, (bb + 1) % 2)

        gd0 = gbufs[0][1]
        for t in range(RB):
            drain_scatter(c00 + kpw - RB + t, (kpw - RB + t) % RB, gd0)

        plsc.subcore_barrier()

        @pl.loop(0, nrows // ZC)
        def _(k):
            pltpu.sync_copy(acc_sh.at[pl.ds(base + k * ZC, ZC)],
                            out_hbm.at[cid, pl.ds(base + k * ZC, ZC)])

    return pl.kernel(
        body,
        out_type=jax.ShapeDtypeStruct((NC, n_nodes, width), jnp.float32),
        mesh=_sc_mesh(),
        scratch_types=[
            pltpu.VMEM((GROUP, CHUNK), jnp.int32),
            pltpu.VMEM((GROUP, CHUNK), jnp.int32),
            pltpu.SemaphoreType.DMA,
            pltpu.VMEM((GROUP, CHUNK), jnp.int32),
            pltpu.VMEM((GROUP, CHUNK), jnp.int32),
            pltpu.SemaphoreType.DMA,
            pltpu.VMEM((CHUNK, width), jnp.float32),
            pltpu.SemaphoreType.DMA,
            pltpu.SemaphoreType.DMA,
            pltpu.VMEM((CHUNK, width), jnp.float32),
            pltpu.SemaphoreType.DMA,
            pltpu.SemaphoreType.DMA,
            pltpu.VMEM((CHUNK, width), jnp.float32),
            pltpu.SemaphoreType.DMA,
            pltpu.SemaphoreType.DMA,
            pltpu.VMEM((CHUNK, width), jnp.float32),
            pltpu.SemaphoreType.DMA,
            pltpu.SemaphoreType.DMA,
            pltpu.VMEM((ZC, width), jnp.float32),
            pltpu.VMEM_SHARED((n_nodes, width), jnp.float32),
        ],
    )(table, src2d, dst2d)


def _tc_blocks(n):
    for nb in (10, 8, 5, 4, 2, 1):
        if n % nb == 0 and (n // nb) % 8 == 0:
            return nb
    return 1


DVW = 8


def _tc_matmul(x, W1):
    n, d = x.shape
    hid = W1.shape[1]
    nb = _tc_blocks(n)
    bs = n // nb

    def body(x_ref, w_ref, o_ref):
        o_ref[...] = jnp.dot(x_ref[...], w_ref[...],
                             preferred_element_type=jnp.float32)

    return pl.pallas_call(
        body,
        grid=(nb,),
        in_specs=[
            pl.BlockSpec((bs, d), lambda i: (i, 0)),
            pl.BlockSpec((d, hid), lambda i: (0, 0)),
        ],
        out_specs=pl.BlockSpec((bs, hid), lambda i: (i, 0)),
        out_shape=jax.ShapeDtypeStruct((n, hid), jnp.float32),
    )(x, W1)


def _tc_scale(h, degA, degB):
    n, hid = h.shape
    nb = _tc_blocks(n)
    bs = n // nb

    def body(h_ref, da_ref, db_ref, o_ref, dv_ref):
        deg = da_ref[:, 0:1] + db_ref[:, 0:1] + 1.0
        dinv = lax.rsqrt(deg)
        o_ref[...] = h_ref[...] * dinv
        dv_ref[...] = jnp.broadcast_to(dinv, (bs, DVW))

    return pl.pallas_call(
        body,
        grid=(nb,),
        in_specs=[
            pl.BlockSpec((bs, hid), lambda i: (i, 0)),
            pl.BlockSpec((bs, DEGW), lambda i: (i, 0)),
            pl.BlockSpec((bs, DEGW), lambda i: (i, 0)),
        ],
        out_specs=[
            pl.BlockSpec((bs, hid), lambda i: (i, 0)),
            pl.BlockSpec((bs, DVW), lambda i: (i, 0)),
        ],
        out_shape=[
            jax.ShapeDtypeStruct((n, hid), jnp.float32),
            jax.ShapeDtypeStruct((n, DVW), jnp.float32),
        ],
    )(h, degA, degB)


def _tc_layer2_pre(accA, accB, hprime, dinv8, b1row):
    n, hid = hprime.shape
    nb = _tc_blocks(n)
    bs = n // nb

    def body(aa_ref, ab_ref, hp_ref, dv_ref, b1_ref, o_ref):
        dinv = dv_ref[:, 0:1]
        agg = aa_ref[...] + ab_ref[...] + hp_ref[...]
        g = jnp.maximum(agg * dinv + b1_ref[...], 0.0)
        o_ref[...] = g * dinv

    return pl.pallas_call(
        body,
        grid=(nb,),
        in_specs=[
            pl.BlockSpec((bs, hid), lambda i: (i, 0)),
            pl.BlockSpec((bs, hid), lambda i: (i, 0)),
            pl.BlockSpec((bs, hid), lambda i: (i, 0)),
            pl.BlockSpec((bs, DVW), lambda i: (i, 0)),
            pl.BlockSpec((1, hid), lambda i: (0, 0)),
        ],
        out_specs=pl.BlockSpec((bs, hid), lambda i: (i, 0)),
        out_shape=jax.ShapeDtypeStruct((n, hid), jnp.float32),
    )(accA, accB, hprime, dinv8, b1row)


def _tc_final(acc2A, acc2B, gprime, dinv8, W2p, b2row):
    n, hid = gprime.shape
    w = W2p.shape[1]
    nb = _tc_blocks(n)
    bs = n // nb

    def body(aa_ref, ab_ref, gp_ref, dv_ref, w_ref, b2_ref, o_ref):
        dinv = dv_ref[:, 0:1]
        agg = aa_ref[...] + ab_ref[...] + gp_ref[...]
        t = jnp.dot(agg, w_ref[...], preferred_element_type=jnp.float32)
        o_ref[...] = t * dinv + b2_ref[...]

    return pl.pallas_call(
        body,
        grid=(nb,),
        in_specs=[
            pl.BlockSpec((bs, hid), lambda i: (i, 0)),
            pl.BlockSpec((bs, hid), lambda i: (i, 0)),
            pl.BlockSpec((bs, hid), lambda i: (i, 0)),
            pl.BlockSpec((bs, DVW), lambda i: (i, 0)),
            pl.BlockSpec((hid, w), lambda i: (0, 0)),
            pl.BlockSpec((1, w), lambda i: (0, 0)),
        ],
        out_specs=pl.BlockSpec((bs, w), lambda i: (i, 0)),
        out_shape=jax.ShapeDtypeStruct((n, w), jnp.float32),
    )(acc2A, acc2B, gprime, dinv8, W2p, b2row)


PADW = 16


def kernel(x, edge_index, W1, b1, W2, b2):
    n = x.shape[0]
    e = edge_index.shape[1]
    d_out = W2.shape[1]
    assert e % CHUNK == 0 and n % ZC == 0

    nchunk = e // CHUNK
    kpw_raw = -(-nchunk // NW)
    kpw = -(-kpw_raw // GROUP) * GROUP
    npad = kpw * NW
    srcp = jnp.pad(edge_index[0], (0, npad * CHUNK - e)).reshape(npad, CHUNK)
    dstp = jnp.pad(edge_index[1], (0, npad * CHUNK - e)).reshape(npad, CHUNK)

    degp = _sc_degree(dstp, n, nchunk)
    degA, degB = degp[0], degp[1]

    h = _tc_matmul(x, W1)
    hp, dinv8 = _tc_scale(h, degA, degB)
    accp = _sc_propagate(hp, srcp, dstp, nchunk)

    gp = _tc_layer2_pre(accp[0], accp[1], hp, dinv8,
                        b1.reshape(1, -1))
    acc2p = _sc_propagate(gp, srcp, dstp, nchunk)

    W2p = jnp.pad(W2, ((0, 0), (0, PADW - d_out)))
    b2p = jnp.pad(b2, (0, PADW - d_out)).reshape(1, PADW)
    outp = _tc_final(acc2p[0], acc2p[1], gp, dinv8, W2p, b2p)
    return outp[:, :d_out]


# speedup vs baseline: 1.1584x; 1.1584x over previous
import jax
import jax.numpy as jnp
from jax import lax
from jax.experimental import pallas as pl
from jax.experimental.pallas import tpu as pltpu
from jax.experimental.pallas import tpu_sc as plsc

NC = 2
NS = 16
NW = NC * NS
CHUNK = 80
DEGW = 128


def _sc_mesh():
    return plsc.VectorSubcoreMesh(
        core_axis_name="c", subcore_axis_name="s",
        num_cores=NC, num_subcores=NS)


def _fill_rows(buf, nrows, width, value):
    v16 = jnp.full((16,), value, jnp.float32)

    @pl.loop(0, nrows)
    def _(r):
        @pl.loop(0, width // 16)
        def _(j):
            buf[r, pl.ds(j * 16, 16)] = v16


ZC = 16


def _tile_rows(n_nodes, sid):
    lo = (n_nodes // NS) // ZC * ZC
    base = sid * lo
    nrows = lo + (n_nodes - NS * lo) * (sid == NS - 1)
    return base, nrows


GROUP = 8


def _sc_degree(dst2d, n_nodes, nchunk):
    nchunk_pad = dst2d.shape[0]
    kpw = nchunk_pad // NW
    ngrp = kpw // GROUP

    def body(dst_hbm, out_hbm, gdst0, gsem0, gdst1, gsem1,
             ones_v, zero_v, acc_sh, ssem):
        cid = lax.axis_index("c")
        sid = lax.axis_index("s")
        wid = cid * NS + sid
        _fill_rows(ones_v, CHUNK, DEGW, 1.0)
        _fill_rows(zero_v, ZC, DEGW, 0.0)
        base, nrows = _tile_rows(n_nodes, sid)

        @pl.loop(0, nrows // ZC)
        def _(k):
            pltpu.async_copy(zero_v, acc_sh.at[pl.ds(base + k * ZC, ZC)],
                             ssem)

        @pl.loop(0, nrows // ZC)
        def _(k):
            pltpu.make_async_copy(zero_v, acc_sh.at[pl.ds(base, ZC)],
                                  ssem).wait()

        plsc.subcore_barrier()

        c00 = wid * kpw
        gbufs = ((gdst0, gsem0), (gdst1, gsem1))

        def fetch(g, b):
            gd, gsem = gbufs[b]

            @pl.when((g < ngrp) & (c00 + g * GROUP < nchunk))
            def _():
                pltpu.async_copy(dst_hbm.at[pl.ds(c00 + g * GROUP, GROUP)],
                                 gd, gsem)

        def wait_fetch(g, b):
            gd, gsem = gbufs[b]

            @pl.when((g < ngrp) & (c00 + g * GROUP < nchunk))
            def _():
                pltpu.make_async_copy(dst_hbm.at[pl.ds(0, GROUP)],
                                      gd, gsem).wait()

        fetch(0, 0)

        @pl.loop(0, ngrp)
        def _(g):
            b = lax.rem(g, 2)
            for bb in range(2):
                @pl.when(b == bb)
                def _():
                    gd, _gsem = gbufs[bb]
                    wait_fetch(g, bb)
                    fetch(g + 1, (bb + 1) % 2)
                    for j in range(GROUP):
                        c = c00 + g * GROUP + j

                        @pl.when(c < nchunk)
                        def _():
                            pltpu.async_copy(ones_v, acc_sh.at[gd.at[j]],
                                             ssem, add=True)
                    for j in range(GROUP):
                        c = c00 + g * GROUP + j

                        @pl.when(c < nchunk)
                        def _():
                            pltpu.make_async_copy(
                                ones_v, acc_sh.at[gd.at[j]], ssem).wait()

        plsc.subcore_barrier()

        lo = (n_nodes // NS) // ZC * ZC
        rem = n_nodes - NS * lo
        pltpu.sync_copy(acc_sh.at[pl.ds(base, lo)],
                        out_hbm.at[cid, pl.ds(base, lo)])
        if rem:
            @pl.when(sid == NS - 1)
            def _():
                pltpu.sync_copy(acc_sh.at[pl.ds(NS * lo, rem)],
                                out_hbm.at[cid, pl.ds(NS * lo, rem)])

    return pl.kernel(
        body,
        out_type=jax.ShapeDtypeStruct((NC, n_nodes, DEGW), jnp.float32),
        mesh=_sc_mesh(),
        scratch_types=[
            pltpu.VMEM((GROUP, CHUNK), jnp.int32),
            pltpu.SemaphoreType.DMA,
            pltpu.VMEM((GROUP, CHUNK), jnp.int32),
            pltpu.SemaphoreType.DMA,
            pltpu.VMEM((CHUNK, DEGW), jnp.float32),
            pltpu.VMEM((ZC, DEGW), jnp.float32),
            pltpu.VMEM_SHARED((n_nodes, DEGW), jnp.float32),
            pltpu.SemaphoreType.DMA,
        ],
    )(dst2d)


def _sc_propagate(table, src2d, dst2d, nchunk):
    n_nodes, width = table.shape
    nchunk_pad = src2d.shape[0]
    kpw = nchunk_pad // NW
    ngrp = kpw // GROUP

    def body(tab_hbm, src_hbm, dst_hbm, out_hbm,
             gsrc0, gdst0, gsem0, gsrc1, gdst1, gsem1,
             rows0, rsem0, ssem0, rows1, rsem1, ssem1,
             rows2, rsem2, ssem2, rows3, rsem3, ssem3,
             zero_v, acc_sh):
        cid = lax.axis_index("c")
        sid = lax.axis_index("s")
        wid = cid * NS + sid
        _fill_rows(zero_v, ZC, width, 0.0)
        base, nrows = _tile_rows(n_nodes, sid)

        @pl.loop(0, nrows // ZC)
        def _(k):
            pltpu.async_copy(zero_v, acc_sh.at[pl.ds(base + k * ZC, ZC)],
                             rsem0)

        @pl.loop(0, nrows // ZC)
        def _(k):
            pltpu.make_async_copy(zero_v, acc_sh.at[pl.ds(base, ZC)],
                                  rsem0).wait()

        plsc.subcore_barrier()

        c00 = wid * kpw
        gbufs = ((gsrc0, gdst0, gsem0), (gsrc1, gdst1, gsem1))
        rbufs = ((rows0, rsem0, ssem0), (rows1, rsem1, ssem1),
                 (rows2, rsem2, ssem2), (rows3, rsem3, ssem3))
        RB = len(rbufs)

        def fetch(g, b):
            gs, gd, gsem = gbufs[b]

            @pl.when((g < ngrp) & (c00 + g * GROUP < nchunk))
            def _():
                pltpu.async_copy(src_hbm.at[pl.ds(c00 + g * GROUP, GROUP)],
                                 gs, gsem)
                pltpu.async_copy(dst_hbm.at[pl.ds(c00 + g * GROUP, GROUP)],
                                 gd, gsem)

        def wait_fetch(g, b):
            gs, gd, gsem = gbufs[b]

            @pl.when((g < ngrp) & (c00 + g * GROUP < nchunk))
            def _():
                pltpu.make_async_copy(src_hbm.at[pl.ds(0, GROUP)],
                                      gs, gsem).wait()
                pltpu.make_async_copy(dst_hbm.at[pl.ds(0, GROUP)],
                                      gd, gsem).wait()

        def drain_scatter(c, rb, gd_any):
            rows, _rsem, ssem = rbufs[rb]

            @pl.when((c >= c00) & (c < nchunk))
            def _():
                pltpu.make_async_copy(rows, acc_sh.at[gd_any.at[0]],
                                      ssem).wait()

        fetch(0, 0)

        @pl.loop(0, ngrp)
        def _(g):
            gb = lax.rem(g, 2)
            for bb in range(2):
                @pl.when(gb == bb)
                def _():
                    gs, gd, _gsem = gbufs[bb]
                    cg0 = c00 + g * GROUP
                    wait_fetch(g, bb)

                    def start(j):
                        c = cg0 + j
                        drain_scatter(c - RB, j % RB, gd)
                        rows, rsem, _ssem = rbufs[j % RB]

                        @pl.when(c < nchunk)
                        def _():
                            pltpu.async_copy(tab_hbm.at[gs.at[j]],
                                             rows, rsem)

                    def finish(j):
                        c = cg0 + j
                        rows, rsem, ssem = rbufs[j % RB]

                        @pl.when(c < nchunk)
                        def _():
                            pltpu.make_async_copy(tab_hbm.at[gs.at[j]],
                                                  rows, rsem).wait()
                            pltpu.async_copy(rows, acc_sh.at[gd.at[j]],
                                             ssem, add=True)

                    start(0)
                    start(1)
                    for j in range(GROUP):
                        if j + 2 < GROUP:
                            start(j + 2)
                        finish(j)
                        if j == 1:
                            fetch(g + 1, (bb + 1) % 2)

        gd0 = gbufs[0][1]
        for t in range(RB):
            drain_scatter(c00 + kpw - RB + t, (kpw - RB + t) % RB, gd0)

        plsc.subcore_barrier()

        lo = (n_nodes // NS) // ZC * ZC
        rem = n_nodes - NS * lo
        pltpu.sync_copy(acc_sh.at[pl.ds(base, lo)],
                        out_hbm.at[cid, pl.ds(base, lo)])
        if rem:
            @pl.when(sid == NS - 1)
            def _():
                pltpu.sync_copy(acc_sh.at[pl.ds(NS * lo, rem)],
                                out_hbm.at[cid, pl.ds(NS * lo, rem)])

    return pl.kernel(
        body,
        out_type=jax.ShapeDtypeStruct((NC, n_nodes, width), jnp.float32),
        mesh=_sc_mesh(),
        scratch_types=[
            pltpu.VMEM((GROUP, CHUNK), jnp.int32),
            pltpu.VMEM((GROUP, CHUNK), jnp.int32),
            pltpu.SemaphoreType.DMA,
            pltpu.VMEM((GROUP, CHUNK), jnp.int32),
            pltpu.VMEM((GROUP, CHUNK), jnp.int32),
            pltpu.SemaphoreType.DMA,
            pltpu.VMEM((CHUNK, width), jnp.float32),
            pltpu.SemaphoreType.DMA,
            pltpu.SemaphoreType.DMA,
            pltpu.VMEM((CHUNK, width), jnp.float32),
            pltpu.SemaphoreType.DMA,
            pltpu.SemaphoreType.DMA,
            pltpu.VMEM((CHUNK, width), jnp.float32),
            pltpu.SemaphoreType.DMA,
            pltpu.SemaphoreType.DMA,
            pltpu.VMEM((CHUNK, width), jnp.float32),
            pltpu.SemaphoreType.DMA,
            pltpu.SemaphoreType.DMA,
            pltpu.VMEM((ZC, width), jnp.float32),
            pltpu.VMEM_SHARED((n_nodes, width), jnp.float32),
        ],
    )(table, src2d, dst2d)


def _tc_blocks(n):
    for nb in (10, 8, 5, 4, 2, 1):
        if n % nb == 0 and (n // nb) % 8 == 0:
            return nb
    return 1


DVW = 8


def _tc_matmul(x, W1):
    n, d = x.shape
    hid = W1.shape[1]
    nb = _tc_blocks(n)
    bs = n // nb

    def body(x_ref, w_ref, o_ref):
        o_ref[...] = jnp.dot(x_ref[...], w_ref[...],
                             preferred_element_type=jnp.float32)

    return pl.pallas_call(
        body,
        grid=(nb,),
        in_specs=[
            pl.BlockSpec((bs, d), lambda i: (i, 0)),
            pl.BlockSpec((d, hid), lambda i: (0, 0)),
        ],
        out_specs=pl.BlockSpec((bs, hid), lambda i: (i, 0)),
        out_shape=jax.ShapeDtypeStruct((n, hid), jnp.float32),
    )(x, W1)


def _tc_scale(h, degA, degB):
    n, hid = h.shape
    nb = _tc_blocks(n)
    bs = n // nb

    def body(h_ref, da_ref, db_ref, o_ref, dv_ref):
        deg = da_ref[:, 0:1] + db_ref[:, 0:1] + 1.0
        dinv = lax.rsqrt(deg)
        o_ref[...] = h_ref[...] * dinv
        dv_ref[...] = jnp.broadcast_to(dinv, (bs, DVW))

    return pl.pallas_call(
        body,
        grid=(nb,),
        in_specs=[
            pl.BlockSpec((bs, hid), lambda i: (i, 0)),
            pl.BlockSpec((bs, DEGW), lambda i: (i, 0)),
            pl.BlockSpec((bs, DEGW), lambda i: (i, 0)),
        ],
        out_specs=[
            pl.BlockSpec((bs, hid), lambda i: (i, 0)),
            pl.BlockSpec((bs, DVW), lambda i: (i, 0)),
        ],
        out_shape=[
            jax.ShapeDtypeStruct((n, hid), jnp.float32),
            jax.ShapeDtypeStruct((n, DVW), jnp.float32),
        ],
    )(h, degA, degB)


def _tc_layer2_pre(accA, accB, hprime, dinv8, b1row):
    n, hid = hprime.shape
    nb = _tc_blocks(n)
    bs = n // nb

    def body(aa_ref, ab_ref, hp_ref, dv_ref, b1_ref, o_ref):
        dinv = dv_ref[:, 0:1]
        agg = aa_ref[...] + ab_ref[...] + hp_ref[...]
        g = jnp.maximum(agg * dinv + b1_ref[...], 0.0)
        o_ref[...] = g * dinv

    return pl.pallas_call(
        body,
        grid=(nb,),
        in_specs=[
            pl.BlockSpec((bs, hid), lambda i: (i, 0)),
            pl.BlockSpec((bs, hid), lambda i: (i, 0)),
            pl.BlockSpec((bs, hid), lambda i: (i, 0)),
            pl.BlockSpec((bs, DVW), lambda i: (i, 0)),
            pl.BlockSpec((1, hid), lambda i: (0, 0)),
        ],
        out_specs=pl.BlockSpec((bs, hid), lambda i: (i, 0)),
        out_shape=jax.ShapeDtypeStruct((n, hid), jnp.float32),
    )(accA, accB, hprime, dinv8, b1row)


def _tc_final(acc2A, acc2B, gprime, dinv8, W2p, b2row):
    n, hid = gprime.shape
    w = W2p.shape[1]
    nb = _tc_blocks(n)
    bs = n // nb

    def body(aa_ref, ab_ref, gp_ref, dv_ref, w_ref, b2_ref, o_ref):
        dinv = dv_ref[:, 0:1]
        agg = aa_ref[...] + ab_ref[...] + gp_ref[...]
        t = jnp.dot(agg, w_ref[...], preferred_element_type=jnp.float32)
        o_ref[...] = t * dinv + b2_ref[...]

    return pl.pallas_call(
        body,
        grid=(nb,),
        in_specs=[
            pl.BlockSpec((bs, hid), lambda i: (i, 0)),
            pl.BlockSpec((bs, hid), lambda i: (i, 0)),
            pl.BlockSpec((bs, hid), lambda i: (i, 0)),
            pl.BlockSpec((bs, DVW), lambda i: (i, 0)),
            pl.BlockSpec((hid, w), lambda i: (0, 0)),
            pl.BlockSpec((1, w), lambda i: (0, 0)),
        ],
        out_specs=pl.BlockSpec((bs, w), lambda i: (i, 0)),
        out_shape=jax.ShapeDtypeStruct((n, w), jnp.float32),
    )(acc2A, acc2B, gprime, dinv8, W2p, b2row)


PADW = 16


def kernel(x, edge_index, W1, b1, W2, b2):
    n = x.shape[0]
    e = edge_index.shape[1]
    d_out = W2.shape[1]
    assert e % CHUNK == 0 and n % ZC == 0

    nchunk = e // CHUNK
    kpw_raw = -(-nchunk // NW)
    kpw = -(-kpw_raw // GROUP) * GROUP
    npad = kpw * NW
    srcp = jnp.pad(edge_index[0], (0, npad * CHUNK - e)).reshape(npad, CHUNK)
    dstp = jnp.pad(edge_index[1], (0, npad * CHUNK - e)).reshape(npad, CHUNK)

    degp = _sc_degree(dstp, n, nchunk)
    degA, degB = degp[0], degp[1]

    h = _tc_matmul(x, W1)
    hp, dinv8 = _tc_scale(h, degA, degB)
    accp = _sc_propagate(hp, srcp, dstp, nchunk)

    gp = _tc_layer2_pre(accp[0], accp[1], hp, dinv8,
                        b1.reshape(1, -1))
    acc2p = _sc_propagate(gp, srcp, dstp, nchunk)

    W2p = jnp.pad(W2, ((0, 0), (0, PADW - d_out)))
    b2p = jnp.pad(b2, (0, PADW - d_out)).reshape(1, PADW)
    outp = _tc_final(acc2p[0], acc2p[1], gp, dinv8, W2p, b2p)
    return outp[:, :d_out]
